# baseline (speedup 1.0000x reference)
import jax
import jax.numpy as jnp
from jax import lax
from jax.experimental import pallas as pl
from jax.experimental.pallas import tpu as pltpu

_K = 9
_INF = float("inf")


def _body(xt_ref, wt1_ref, b1_ref, g1_ref, be1_ref, m1_ref, v1_ref,
          wuv_ref, be_ref, wp2_ref, bp2_ref, out_ref):
    N = xt_ref.shape[1]
    C2 = be_ref.shape[1]

    y = jnp.dot(xt_ref[0], wt1_ref[...], preferred_element_type=jnp.float32)
    y = y + b1_ref[...]
    xf = (y - m1_ref[...]) / jnp.sqrt(v1_ref[...] + 1e-5) * g1_ref[...] \
        + be1_ref[...]
    uv = jnp.dot(xf, wuv_ref[...], preferred_element_type=jnp.float32)
    u = uv[:, :C2] + be_ref[...]
    v = uv[:, C2:]

    sq = jnp.sum(xf * xf, axis=1, keepdims=True)
    gram = lax.dot_general(xf, xf, (((1,), (1,)), ((), ())),
                           preferred_element_type=jnp.float32)
    d = (sq + jnp.transpose(sq)) - 2.0 * gram

    colf = lax.broadcasted_iota(jnp.int32, (N, N), 1).astype(jnp.float32)
    bigf = jnp.float32(N)
    vmn = jnp.full((N, C2), _INF, jnp.float32)
    vmx = jnp.full((N, C2), -_INF, jnp.float32)
    for _ in range(_K):
        mn = jnp.min(d, axis=1, keepdims=True)
        eq = d <= mn
        midx = jnp.min(jnp.where(eq, colf, bigf), axis=1, keepdims=True)
        mask = colf == midx
        vrow = jnp.dot(mask.astype(jnp.bfloat16), v.astype(jnp.bfloat16),
                       preferred_element_type=jnp.float32)
        vmn = jnp.minimum(vmn, vrow)
        vmx = jnp.maximum(vmx, vrow)
        d = jnp.where(mask, _INF, d)
    h = jnp.maximum(jax.nn.gelu(u + vmn), jax.nn.gelu(u + vmx))

    z = jnp.dot(h, wp2_ref[...], preferred_element_type=jnp.float32)
    out_ref[0] = z + bp2_ref[...]


def kernel(x, W_fc1, b_fc1, bn1_gamma, bn1_beta, bn1_mean, bn1_var,
           W_edge, b_edge, W_fc2, b_fc2, bn2_gamma, bn2_beta, bn2_mean,
           bn2_var):
    eps = 1e-5
    B, C, H, W = x.shape
    N = H * W
    C2 = 2 * C

    A = W_edge[:, :C]
    Bm = W_edge[:, C:]
    wuv = jnp.concatenate([(A - Bm).T, Bm.T], axis=1)
    be = b_edge[None, :]

    s2 = bn2_gamma / jnp.sqrt(bn2_var + eps)
    wp2 = (W_fc2 * s2[:, None]).T
    bp2 = ((b_fc2 - bn2_mean) * s2 + bn2_beta)[None, :]

    xt = x.reshape(B, C, N).transpose(0, 2, 1)

    row = lambda a: a[None, :]
    out = pl.pallas_call(
        _body,
        grid=(B,),
        in_specs=[
            pl.BlockSpec((1, N, C), lambda b: (b, 0, 0)),
            pl.BlockSpec((C, C), lambda b: (0, 0)),
            pl.BlockSpec((1, C), lambda b: (0, 0)),
            pl.BlockSpec((1, C), lambda b: (0, 0)),
            pl.BlockSpec((1, C), lambda b: (0, 0)),
            pl.BlockSpec((1, C), lambda b: (0, 0)),
            pl.BlockSpec((1, C), lambda b: (0, 0)),
            pl.BlockSpec((C, 2 * C2), lambda b: (0, 0)),
            pl.BlockSpec((1, C2), lambda b: (0, 0)),
            pl.BlockSpec((C2, C), lambda b: (0, 0)),
            pl.BlockSpec((1, C), lambda b: (0, 0)),
        ],
        out_specs=pl.BlockSpec((1, N, C), lambda b: (b, 0, 0)),
        out_shape=jax.ShapeDtypeStruct((B, N, C), jnp.float32),
        compiler_params=pltpu.CompilerParams(
            dimension_semantics=("parallel",),
        ),
    )(xt, W_fc1.T, row(b_fc1), row(bn1_gamma), row(bn1_beta),
      row(bn1_mean), row(bn1_var), wuv, be, wp2, bp2)

    return out.transpose(0, 2, 1).reshape(B, C, H, W)


# speedup vs baseline: 1.0149x; 1.0149x over previous
import jax
import jax.numpy as jnp
from jax import lax
from jax.experimental import pallas as pl
from jax.experimental.pallas import tpu as pltpu

_K = 9
_INF = float("inf")


def _body(xt_ref, wt1_ref, b1_ref, g1_ref, be1_ref, m1_ref, v1_ref,
          wuv_ref, be_ref, wp2_ref, bp2_ref, out_ref):
    N = xt_ref.shape[1]
    C2 = be_ref.shape[1]

    y = jnp.dot(xt_ref[0], wt1_ref[...], preferred_element_type=jnp.float32)
    y = y + b1_ref[...]
    xf = (y - m1_ref[...]) / jnp.sqrt(v1_ref[...] + 1e-5) * g1_ref[...] \
        + be1_ref[...]
    uv = jnp.dot(xf, wuv_ref[...], preferred_element_type=jnp.float32)
    u = uv[:, :C2] + be_ref[...]
    v = uv[:, C2:]

    sq = jnp.sum(xf * xf, axis=1, keepdims=True)
    gram = lax.dot_general(xf, xf, (((1,), (1,)), ((), ())),
                           preferred_element_type=jnp.float32)
    d = (sq + jnp.transpose(sq)) - 2.0 * gram

    colf = lax.broadcasted_iota(jnp.int32, (N, N), 1).astype(jnp.float32)
    bigf = jnp.float32(N)
    vmn = jnp.full((N, C2), _INF, jnp.float32)
    vmx = jnp.full((N, C2), -_INF, jnp.float32)
    for _ in range(_K):
        mn = jnp.min(d, axis=1, keepdims=True)
        midx = jnp.min(jnp.where(d <= mn, colf, bigf), axis=1, keepdims=True)
        mask = colf == midx
        vrow = jnp.dot(mask.astype(jnp.float32), v,
                       preferred_element_type=jnp.float32)
        vmn = jnp.minimum(vmn, vrow)
        vmx = jnp.maximum(vmx, vrow)
        d = jnp.where(mask, _INF, d)
    h = jnp.maximum(jax.nn.gelu(u + vmn), jax.nn.gelu(u + vmx))

    z = jnp.dot(h, wp2_ref[...], preferred_element_type=jnp.float32)
    out_ref[0] = z + bp2_ref[...]


def kernel(x, W_fc1, b_fc1, bn1_gamma, bn1_beta, bn1_mean, bn1_var,
           W_edge, b_edge, W_fc2, b_fc2, bn2_gamma, bn2_beta, bn2_mean,
           bn2_var):
    eps = 1e-5
    B, C, H, W = x.shape
    N = H * W
    C2 = 2 * C

    A = W_edge[:, :C]
    Bm = W_edge[:, C:]
    wuv = jnp.concatenate([(A - Bm).T, Bm.T], axis=1)
    be = b_edge[None, :]

    s2 = bn2_gamma / jnp.sqrt(bn2_var + eps)
    wp2 = (W_fc2 * s2[:, None]).T
    bp2 = ((b_fc2 - bn2_mean) * s2 + bn2_beta)[None, :]

    xt = x.reshape(B, C, N).transpose(0, 2, 1)

    row = lambda a: a[None, :]
    out = pl.pallas_call(
        _body,
        grid=(B,),
        in_specs=[
            pl.BlockSpec((1, N, C), lambda b: (b, 0, 0)),
            pl.BlockSpec((C, C), lambda b: (0, 0)),
            pl.BlockSpec((1, C), lambda b: (0, 0)),
            pl.BlockSpec((1, C), lambda b: (0, 0)),
            pl.BlockSpec((1, C), lambda b: (0, 0)),
            pl.BlockSpec((1, C), lambda b: (0, 0)),
            pl.BlockSpec((1, C), lambda b: (0, 0)),
            pl.BlockSpec((C, 2 * C2), lambda b: (0, 0)),
            pl.BlockSpec((1, C2), lambda b: (0, 0)),
            pl.BlockSpec((C2, C), lambda b: (0, 0)),
            pl.BlockSpec((1, C), lambda b: (0, 0)),
        ],
        out_specs=pl.BlockSpec((1, N, C), lambda b: (b, 0, 0)),
        out_shape=jax.ShapeDtypeStruct((B, N, C), jnp.float32),
        compiler_params=pltpu.CompilerParams(
            dimension_semantics=("parallel",),
        ),
    )(xt, W_fc1.T, row(b_fc1), row(bn1_gamma), row(bn1_beta),
      row(bn1_mean), row(bn1_var), wuv, be, wp2, bp2)

    return out.transpose(0, 2, 1).reshape(B, C, H, W)


# speedup vs baseline: 1.0189x; 1.0039x over previous
import jax
import jax.numpy as jnp
from jax import lax
from jax.experimental import pallas as pl
from jax.experimental.pallas import tpu as pltpu

_K = 9
_INF = float("inf")


def _body(xt_ref, wt1_ref, b1_ref, g1_ref, be1_ref, m1_ref, v1_ref,
          wuv_ref, be_ref, wp2_ref, bp2_ref, out_ref):
    for i in range(xt_ref.shape[0]):
        _one_image(i, xt_ref, wt1_ref, b1_ref, g1_ref, be1_ref, m1_ref,
                   v1_ref, wuv_ref, be_ref, wp2_ref, bp2_ref, out_ref)


def _one_image(i, xt_ref, wt1_ref, b1_ref, g1_ref, be1_ref, m1_ref, v1_ref,
               wuv_ref, be_ref, wp2_ref, bp2_ref, out_ref):
    N = xt_ref.shape[1]
    C2 = be_ref.shape[1]

    y = jnp.dot(xt_ref[i], wt1_ref[...], preferred_element_type=jnp.float32)
    y = y + b1_ref[...]
    xf = (y - m1_ref[...]) / jnp.sqrt(v1_ref[...] + 1e-5) * g1_ref[...] \
        + be1_ref[...]
    uv = jnp.dot(xf, wuv_ref[...], preferred_element_type=jnp.float32)
    u = uv[:, :C2] + be_ref[...]
    v = uv[:, C2:]

    sq = jnp.sum(xf * xf, axis=1, keepdims=True)
    gram = lax.dot_general(xf, xf, (((1,), (1,)), ((), ())),
                           preferred_element_type=jnp.float32)
    d = (sq + jnp.transpose(sq)) - 2.0 * gram

    colf = lax.broadcasted_iota(jnp.int32, (N, N), 1).astype(jnp.float32)
    bigf = jnp.float32(N)
    vmn = jnp.full((N, C2), _INF, jnp.float32)
    vmx = jnp.full((N, C2), -_INF, jnp.float32)
    for _ in range(_K):
        mn = jnp.min(d, axis=1, keepdims=True)
        midx = jnp.min(jnp.where(d <= mn, colf, bigf), axis=1, keepdims=True)
        mask = colf == midx
        vrow = jnp.dot(mask.astype(jnp.float32), v,
                       preferred_element_type=jnp.float32)
        vmn = jnp.minimum(vmn, vrow)
        vmx = jnp.maximum(vmx, vrow)
        d = jnp.where(mask, _INF, d)
    h = jnp.maximum(jax.nn.gelu(u + vmn), jax.nn.gelu(u + vmx))

    z = jnp.dot(h, wp2_ref[...], preferred_element_type=jnp.float32)
    out_ref[i] = z + bp2_ref[...]


def kernel(x, W_fc1, b_fc1, bn1_gamma, bn1_beta, bn1_mean, bn1_var,
           W_edge, b_edge, W_fc2, b_fc2, bn2_gamma, bn2_beta, bn2_mean,
           bn2_var):
    eps = 1e-5
    B, C, H, W = x.shape
    N = H * W
    C2 = 2 * C

    A = W_edge[:, :C]
    Bm = W_edge[:, C:]
    wuv = jnp.concatenate([(A - Bm).T, Bm.T], axis=1)
    be = b_edge[None, :]

    s2 = bn2_gamma / jnp.sqrt(bn2_var + eps)
    wp2 = (W_fc2 * s2[:, None]).T
    bp2 = ((b_fc2 - bn2_mean) * s2 + bn2_beta)[None, :]

    xt = x.reshape(B, C, N).transpose(0, 2, 1)

    row = lambda a: a[None, :]
    out = pl.pallas_call(
        _body,
        grid=(B // 2,),
        in_specs=[
            pl.BlockSpec((2, N, C), lambda b: (b, 0, 0)),
            pl.BlockSpec((C, C), lambda b: (0, 0)),
            pl.BlockSpec((1, C), lambda b: (0, 0)),
            pl.BlockSpec((1, C), lambda b: (0, 0)),
            pl.BlockSpec((1, C), lambda b: (0, 0)),
            pl.BlockSpec((1, C), lambda b: (0, 0)),
            pl.BlockSpec((1, C), lambda b: (0, 0)),
            pl.BlockSpec((C, 2 * C2), lambda b: (0, 0)),
            pl.BlockSpec((1, C2), lambda b: (0, 0)),
            pl.BlockSpec((C2, C), lambda b: (0, 0)),
            pl.BlockSpec((1, C), lambda b: (0, 0)),
        ],
        out_specs=pl.BlockSpec((2, N, C), lambda b: (b, 0, 0)),
        out_shape=jax.ShapeDtypeStruct((B, N, C), jnp.float32),
        compiler_params=pltpu.CompilerParams(
            dimension_semantics=("parallel",),
        ),
    )(xt, W_fc1.T, row(b_fc1), row(bn1_gamma), row(bn1_beta),
      row(bn1_mean), row(bn1_var), wuv, be, wp2, bp2)

    return out.transpose(0, 2, 1).reshape(B, C, H, W)
